# baseline (speedup 1.0000x reference)
import functools

import jax
import jax.numpy as jnp
import numpy as np
from jax import lax
from jax.experimental import pallas as pl
from jax.experimental.pallas import tpu as pltpu
from jax.experimental.pallas import tpu_sc as plsc

VOCAB = 100000
D_MODEL = 64
MAX_LEN = 200
BATCH = 4096

_NC = 2
_NS = 16
_NW = _NC * _NS

_BW = BATCH // _NW
_NBUF = 5
_NLANE = D_MODEL // 16
_DT = D_MODEL // 8
_TW = 8 * _BW
_PADW = _BW + 1


def _pos_embedding_np():
    pos = np.arange(MAX_LEN, dtype=np.float64)[:, None]
    i = np.arange(D_MODEL, dtype=np.float64)[None, :]
    angle = pos / np.power(10000.0, 2.0 * i / D_MODEL)
    pe = np.where((np.arange(D_MODEL)[None, :] % 2) == 0,
                  np.cos(angle), np.sin(angle))
    return pe.astype(np.float32)


_PE_NP = _pos_embedding_np()


def _body(x4_hbm, table_hbm, pe_hbm, out_hbm, *scratch):
    idxb = scratch[0:_NBUF]
    rowsb = scratch[_NBUF:2 * _NBUF]
    blockb = scratch[2 * _NBUF:3 * _NBUF]
    pad_v = scratch[3 * _NBUF]
    pe_v = scratch[3 * _NBUF + 1]
    isem = scratch[3 * _NBUF + 2:3 * _NBUF + 2 + _NBUF]
    gsem = scratch[3 * _NBUF + 2 + _NBUF:3 * _NBUF + 2 + 2 * _NBUF]
    ssem = scratch[3 * _NBUF + 2 + 2 * _NBUF:]

    wid = lax.axis_index("s") * _NC + lax.axis_index("c")

    pltpu.sync_copy(pe_hbm, pe_v)

    def fire_idx(t, b):
        pltpu.async_copy(x4_hbm.at[t // 8, wid, t % 8], idxb[b], isem[b])

    def fire_gather(t, b):
        pltpu.make_async_copy(x4_hbm.at[0, 0, 0], idxb[b], isem[b]).wait()
        pltpu.async_copy(table_hbm.at[idxb[b]], rowsb[b], gsem[b])

    def drain_store(b):
        for _ in range(_DT):
            pltpu.make_async_copy(
                blockb[b].at[pl.ds(0, _TW)], out_hbm.at[0, 0, 0],
                ssem[b]).wait()

    fire_idx(0, 0)
    fire_idx(1, 1)
    fire_idx(2, 2)
    fire_gather(0, 0)
    fire_gather(1, 1)

    lane = lax.iota(jnp.int32, 16)
    rowoff = [(lane + c * 16) * _PADW for c in range(_NLANE)]

    def step(t0):
        for b0 in range(_NBUF):
            t = t0 + b0
            b = b0
            bi = (b0 + 2) % _NBUF
            bg = (b0 + 1) % _NBUF

            bii = (b0 + 3) % _NBUF

            @pl.when(t < MAX_LEN - 3)
            def _():
                fire_idx(t + 3, bii)

            @pl.when(t < MAX_LEN - 2)
            def _():
                fire_gather(t + 2, bi)

            @pl.when(t >= _NBUF - 2)
            def _():
                drain_store(bi)

            pltpu.make_async_copy(
                table_hbm.at[idxb[b]], rowsb[b], gsem[b]).wait()

            pes = [pe_v[t, pl.ds(c * 16, 16)] for c in range(_NLANE)]

            @plsc.parallel_loop(0, _BW, unroll=8,
                                carry=tuple(pes) + tuple(rowoff))
            def _(i, carry):
                for c in range(_NLANE):
                    v = rowsb[b][i, pl.ds(c * 16, 16)] + carry[c]
                    plsc.store_scatter(pad_v, [carry[_NLANE + c] + i], v)
                return carry

            @plsc.parallel_loop(0, D_MODEL, unroll=8)
            def _(d):
                for j in range(_BW // 16):
                    blockb[b][pl.ds(d * _BW + j * 16, 16)] = (
                        pad_v[pl.ds(d * _PADW + j * 16, 16)])

            for dt in range(_DT):
                pltpu.async_copy(
                    blockb[b].at[pl.ds(dt * _TW, _TW)],
                    out_hbm.at[t, dt, wid], ssem[b])

    pl.loop(0, MAX_LEN, step=_NBUF)(step)

    for t in range(MAX_LEN - (_NBUF - 2), MAX_LEN):
        drain_store(t % _NBUF)


_pallas_fn = functools.partial(
    pl.kernel,
    out_type=jax.ShapeDtypeStruct((MAX_LEN, _DT, _NW, _TW), jnp.float32),
    mesh=plsc.VectorSubcoreMesh(
        core_axis_name="c", subcore_axis_name="s",
        num_cores=_NC, num_subcores=_NS,
    ),
    scratch_types=(
        [pltpu.VMEM((_BW,), jnp.int32)] * _NBUF
        + [pltpu.VMEM((_BW, D_MODEL), jnp.float32)] * _NBUF
        + [pltpu.VMEM((D_MODEL * _BW,), jnp.float32)] * _NBUF
        + [pltpu.VMEM((D_MODEL * _PADW,), jnp.float32)]
        + [pltpu.VMEM((MAX_LEN, D_MODEL), jnp.float32)]
        + [pltpu.SemaphoreType.DMA] * (3 * _NBUF)
    ),
    compiler_params=pltpu.CompilerParams(
        use_tc_tiling_on_sc=False, needs_layout_passes=False),
)(_body)


@jax.jit
def kernel(X, table):
    x4 = X.T.reshape(MAX_LEN // 8, 8, _NW, _BW).transpose(0, 2, 1, 3)
    out5 = _pallas_fn(x4, table, jnp.asarray(_PE_NP))
    out = out5.reshape(MAX_LEN, _DT, _NW, 8, _BW)
    out = out.transpose(2, 4, 0, 1, 3)
    return out.reshape(BATCH, MAX_LEN, D_MODEL)


# speedup vs baseline: 1.0082x; 1.0082x over previous
import functools

import jax
import jax.numpy as jnp
import numpy as np
from jax import lax
from jax.experimental import pallas as pl
from jax.experimental.pallas import tpu as pltpu
from jax.experimental.pallas import tpu_sc as plsc

VOCAB = 100000
D_MODEL = 64
MAX_LEN = 200
BATCH = 4096

_NC = 2
_NS = 16
_NW = _NC * _NS

_BW = BATCH // _NW
_NBUF = 4
_NLANE = D_MODEL // 16
_DT = D_MODEL // 8
_TW = 8 * _BW
_PADW = _BW + 1


def _pos_embedding_np():
    pos = np.arange(MAX_LEN, dtype=np.float64)[:, None]
    i = np.arange(D_MODEL, dtype=np.float64)[None, :]
    angle = pos / np.power(10000.0, 2.0 * i / D_MODEL)
    pe = np.where((np.arange(D_MODEL)[None, :] % 2) == 0,
                  np.cos(angle), np.sin(angle))
    return pe.astype(np.float32)


_PE_NP = _pos_embedding_np()


def _body(x4_hbm, table_hbm, pe_hbm, out_hbm, *scratch):
    idxb = scratch[0:_NBUF]
    rowsb = scratch[_NBUF:2 * _NBUF]
    blockb = scratch[2 * _NBUF:3 * _NBUF]
    pad_v = scratch[3 * _NBUF]
    pe_v = scratch[3 * _NBUF + 1]
    isem = scratch[3 * _NBUF + 2:3 * _NBUF + 2 + _NBUF]
    gsem = scratch[3 * _NBUF + 2 + _NBUF:3 * _NBUF + 2 + 2 * _NBUF]
    ssem = scratch[3 * _NBUF + 2 + 2 * _NBUF:]

    wid = lax.axis_index("s") * _NC + lax.axis_index("c")

    pltpu.sync_copy(pe_hbm, pe_v)

    def fire_idx(t, b):
        pltpu.async_copy(x4_hbm.at[t // 8, wid, t % 8], idxb[b], isem[b])

    def fire_gather(t, b):
        pltpu.make_async_copy(x4_hbm.at[0, 0, 0], idxb[b], isem[b]).wait()
        pltpu.async_copy(table_hbm.at[idxb[b]], rowsb[b], gsem[b])

    def drain_store(b):
        pltpu.make_async_copy(
            blockb[b], out_hbm.at[0, pl.ds(0, _DT), 0], ssem[b]).wait()

    fire_idx(0, 0)
    fire_idx(1, 1)
    fire_idx(2, 2)
    fire_gather(0, 0)
    fire_gather(1, 1)

    lane = lax.iota(jnp.int32, 16)
    rowoff = [(lane + c * 16) * _PADW for c in range(_NLANE)]

    def step(t0):
        for b0 in range(_NBUF):
            t = t0 + b0
            b = b0
            bi = (b0 + 2) % _NBUF
            bg = (b0 + 1) % _NBUF

            bii = (b0 + 3) % _NBUF

            @pl.when(t < MAX_LEN - 3)
            def _():
                fire_idx(t + 3, bii)

            @pl.when(t < MAX_LEN - 2)
            def _():
                fire_gather(t + 2, bi)

            @pl.when(t >= 2)
            def _():
                drain_store(bi)

            pltpu.make_async_copy(
                table_hbm.at[idxb[b]], rowsb[b], gsem[b]).wait()

            pes = [pe_v[t, pl.ds(c * 16, 16)] for c in range(_NLANE)]

            @plsc.parallel_loop(0, _BW, unroll=8,
                                carry=tuple(pes) + tuple(rowoff))
            def _(i, carry):
                for c in range(_NLANE):
                    v = rowsb[b][i, pl.ds(c * 16, 16)] + carry[c]
                    plsc.store_scatter(pad_v, [carry[_NLANE + c] + i], v)
                return carry

            @plsc.parallel_loop(0, D_MODEL, unroll=8)
            def _(d):
                dt = d // 8
                r = d - dt * 8
                for j in range(_BW // 16):
                    blockb[b][dt, pl.ds(r * _BW + j * 16, 16)] = (
                        pad_v[pl.ds(d * _PADW + j * 16, 16)])

            pltpu.async_copy(
                blockb[b], out_hbm.at[t, pl.ds(0, _DT), wid], ssem[b])

    pl.loop(0, MAX_LEN, step=_NBUF)(step)

    for t in (MAX_LEN - 2, MAX_LEN - 1):
        drain_store(t % _NBUF)


_pallas_fn = functools.partial(
    pl.kernel,
    out_type=jax.ShapeDtypeStruct((MAX_LEN, _DT, _NW, _TW), jnp.float32),
    mesh=plsc.VectorSubcoreMesh(
        core_axis_name="c", subcore_axis_name="s",
        num_cores=_NC, num_subcores=_NS,
    ),
    scratch_types=(
        [pltpu.VMEM((_BW,), jnp.int32)] * _NBUF
        + [pltpu.VMEM((_BW, D_MODEL), jnp.float32)] * _NBUF
        + [pltpu.VMEM((_DT, _TW), jnp.float32)] * _NBUF
        + [pltpu.VMEM((D_MODEL * _PADW,), jnp.float32)]
        + [pltpu.VMEM((MAX_LEN, D_MODEL), jnp.float32)]
        + [pltpu.SemaphoreType.DMA] * (3 * _NBUF)
    ),
    compiler_params=pltpu.CompilerParams(
        use_tc_tiling_on_sc=False, needs_layout_passes=False),
)(_body)


@jax.jit
def kernel(X, table):
    x4 = X.T.reshape(MAX_LEN // 8, 8, _NW, _BW).transpose(0, 2, 1, 3)
    out5 = _pallas_fn(x4, table, jnp.asarray(_PE_NP))
    out = out5.reshape(MAX_LEN, _DT, _NW, 8, _BW)
    out = out.transpose(2, 4, 0, 1, 3)
    return out.reshape(BATCH, MAX_LEN, D_MODEL)
